# baseline (speedup 1.0000x reference)
import functools

import jax
import jax.numpy as jnp
import numpy as np
from jax import lax
from jax.experimental import pallas as pl
from jax.experimental.pallas import tpu as pltpu
from jax.experimental.pallas import tpu_sc as plsc

_N_VOCAB = 100000
_D = 768
_B = 4
_S = 2048
_LANES = 16

_INFO = plsc.get_sparse_core_info()
_NC = _INFO.num_cores
_NS = _INFO.num_subcores
_NW = _NC * _NS
_ROWS = _B * _S
_S_PER_W = _S // _NW
_CH = 32
_N_CH = (_B * _S_PER_W) // _CH
_H = _N_CH // _B
_NBUF = 3


def _pe_table() -> jnp.ndarray:
    pos = np.arange(_S, dtype=np.float64)[:, None]
    i = np.arange(_D, dtype=np.float64)[None, :]
    angle = pos / np.power(10000.0, (2.0 * np.floor(i / 2.0)) / _D)
    pe = np.where((np.arange(_D)[None, :] % 2) == 0, np.sin(angle), np.cos(angle))
    return jnp.asarray(pe, dtype=jnp.float32)


_MESH = plsc.VectorSubcoreMesh(core_axis_name="c", subcore_axis_name="s")


@functools.partial(
    pl.kernel,
    mesh=_MESH,
    out_type=jax.ShapeDtypeStruct((_ROWS, _D), jnp.float32),
    scratch_types=[
        pltpu.VMEM((_B, _S_PER_W), jnp.int32),
        pltpu.VMEM((_S_PER_W, _D), jnp.float32),
        pltpu.VMEM((_NBUF, _CH, _D), jnp.float32),
        pltpu.SemaphoreType.DMA((_NBUF,)),
        pltpu.SemaphoreType.DMA((_NBUF,)),
        pltpu.SemaphoreType.DMA,
        pltpu.SemaphoreType.DMA,
    ],
)
def _embed_pe(idx_hbm, table_hbm, pe_hbm, out_hbm, idx_v, pe_v, gat_v,
              gat_sem, out_sem, pe_sem, idx_sem):
    wid = lax.axis_index("s") * _NC + lax.axis_index("c")
    s0 = wid * _S_PER_W

    idx_copies = [
        pltpu.async_copy(idx_hbm.at[b, pl.ds(s0, _S_PER_W)], idx_v.at[b],
                         idx_sem)
        for b in range(_B)
    ]
    pe_copy = pltpu.async_copy(pe_hbm.at[pl.ds(s0, _S_PER_W)], pe_v, pe_sem)
    for cp in idx_copies:
        cp.wait()

    def _issue_gather(c):
        b = c // _H
        h = c - b * _H
        p = c % _NBUF
        pltpu.async_copy(
            table_hbm.at[idx_v.at[b, pl.ds(h * _CH, _CH)]],
            gat_v.at[p], gat_sem.at[p],
        )

    def _out_slice(c):
        b = c // _H
        h = c - b * _H
        row0 = b * _S + s0 + h * _CH
        return out_hbm.at[pl.ds(row0, _CH)]

    for c in range(min(_NBUF, _N_CH)):
        _issue_gather(c)

    def _chunk(c, carry):
        p = c % _NBUF
        h = c - (c // _H) * _H

        pltpu.make_async_copy(
            table_hbm.at[idx_v.at[0, pl.ds(0, _CH)]], gat_v.at[p],
            gat_sem.at[p],
        ).wait()

        @pl.when(c == 0)
        def _wait_pe():
            pltpu.make_async_copy(
                pe_hbm.at[pl.ds(s0, _S_PER_W)], pe_v, pe_sem
            ).wait()

        @plsc.parallel_loop(0, _CH, unroll=3)
        def _add_row(r):
            for j in range(_D // _LANES):
                sl = pl.ds(j * _LANES, _LANES)
                plsc.addupdate(gat_v.at[p, r, sl], pe_v[h * _CH + r, sl])

        pltpu.async_copy(gat_v.at[p], _out_slice(c), out_sem.at[p])

        @pl.when(jnp.logical_and(0 < c, c + _NBUF - 1 < _N_CH))
        def _refill():
            pm = (c - 1) % _NBUF
            pltpu.make_async_copy(
                gat_v.at[pm], _out_slice(c - 1), out_sem.at[pm]
            ).wait()
            _issue_gather(c + _NBUF - 1)
        return carry

    lax.fori_loop(0, _N_CH, _chunk, 0)
    for c in range(max(_N_CH - _NBUF, 0), _N_CH):
        pltpu.make_async_copy(
            gat_v.at[c % _NBUF], _out_slice(c), out_sem.at[c % _NBUF]
        ).wait()


def kernel(input, embed_table):
    idx = input.reshape(_B, _S).astype(jnp.int32)
    pe = _pe_table()
    out = _embed_pe(idx, embed_table, pe)
    return out.reshape(_B, _S, _D)


# speedup vs baseline: 1.0125x; 1.0125x over previous
import functools

import jax
import jax.numpy as jnp
import numpy as np
from jax import lax
from jax.experimental import pallas as pl
from jax.experimental.pallas import tpu as pltpu
from jax.experimental.pallas import tpu_sc as plsc

_N_VOCAB = 100000
_D = 768
_B = 4
_S = 2048
_LANES = 16

_INFO = plsc.get_sparse_core_info()
_NC = _INFO.num_cores
_NS = _INFO.num_subcores
_NW = _NC * _NS
_ROWS = _B * _S
_S_PER_W = _S // _NW
_CH = 32
_N_CH = (_B * _S_PER_W) // _CH
_H = _N_CH // _B
_NBUF = 3


def _pe_table() -> jnp.ndarray:
    pos = np.arange(_S, dtype=np.float64)[:, None]
    i = np.arange(_D, dtype=np.float64)[None, :]
    angle = pos / np.power(10000.0, (2.0 * np.floor(i / 2.0)) / _D)
    pe = np.where((np.arange(_D)[None, :] % 2) == 0, np.sin(angle), np.cos(angle))
    return jnp.asarray(pe, dtype=jnp.float32)


_MESH = plsc.VectorSubcoreMesh(core_axis_name="c", subcore_axis_name="s")


@functools.partial(
    pl.kernel,
    mesh=_MESH,
    out_type=jax.ShapeDtypeStruct((_ROWS, _D), jnp.float32),
    scratch_types=[
        pltpu.VMEM((_B, _S_PER_W), jnp.int32),
        pltpu.VMEM((_S_PER_W, _D), jnp.float32),
        pltpu.VMEM((_NBUF, _CH, _D), jnp.float32),
        pltpu.SemaphoreType.DMA((_NBUF,)),
        pltpu.SemaphoreType.DMA((_NBUF,)),
        pltpu.SemaphoreType.DMA,
        pltpu.SemaphoreType.DMA,
    ],
)
def _embed_pe(idx_hbm, table_hbm, pe_hbm, out_hbm, idx_v, pe_v, gat_v,
              gat_sem, out_sem, pe_sem, idx_sem):
    wid = lax.axis_index("s") * _NC + lax.axis_index("c")
    s0 = wid * _S_PER_W

    idx_copies = [
        pltpu.async_copy(idx_hbm.at[b, pl.ds(s0, _S_PER_W)], idx_v.at[b],
                         idx_sem)
        for b in range(_B)
    ]
    pe_copy = pltpu.async_copy(pe_hbm.at[pl.ds(s0, _S_PER_W)], pe_v, pe_sem)
    for cp in idx_copies:
        cp.wait()

    def _issue_gather(c):
        b = c // _H
        h = c - b * _H
        p = c % _NBUF
        pltpu.async_copy(
            table_hbm.at[idx_v.at[b, pl.ds(h * _CH, _CH)]],
            gat_v.at[p], gat_sem.at[p],
        )

    def _out_slice(c):
        b = c // _H
        h = c - b * _H
        row0 = b * _S + s0 + h * _CH
        return out_hbm.at[pl.ds(row0, _CH)]

    for c in range(min(_NBUF, _N_CH)):
        _issue_gather(c)

    def _chunk(c, carry):
        p = c % _NBUF
        h = c - (c // _H) * _H

        pltpu.make_async_copy(
            table_hbm.at[idx_v.at[0, pl.ds(0, _CH)]], gat_v.at[p],
            gat_sem.at[p],
        ).wait()

        @pl.when(c == 0)
        def _wait_pe():
            pltpu.make_async_copy(
                pe_hbm.at[pl.ds(s0, _S_PER_W)], pe_v, pe_sem
            ).wait()

        @plsc.parallel_loop(0, _CH, unroll=2)
        def _add_row(r):
            for j in range(_D // _LANES):
                sl = pl.ds(j * _LANES, _LANES)
                plsc.addupdate(gat_v.at[p, r, sl], pe_v[h * _CH + r, sl])

        pltpu.async_copy(gat_v.at[p], _out_slice(c), out_sem.at[p])

        @pl.when(jnp.logical_and(0 < c, c + _NBUF - 1 < _N_CH))
        def _refill():
            pm = (c - 1) % _NBUF
            pltpu.make_async_copy(
                gat_v.at[pm], _out_slice(c - 1), out_sem.at[pm]
            ).wait()
            _issue_gather(c + _NBUF - 1)
        return carry

    lax.fori_loop(0, _N_CH, _chunk, 0)
    for c in range(max(_N_CH - _NBUF, 0), _N_CH):
        pltpu.make_async_copy(
            gat_v.at[c % _NBUF], _out_slice(c), out_sem.at[c % _NBUF]
        ).wait()


def kernel(input, embed_table):
    idx = input.reshape(_B, _S).astype(jnp.int32)
    pe = _pe_table()
    out = _embed_pe(idx, embed_table, pe)
    return out.reshape(_B, _S, _D)
